# baseline (speedup 1.0000x reference)
import functools

import jax
import jax.numpy as jnp
from jax import lax
from jax.experimental import pallas as pl
from jax.experimental.pallas import tpu as pltpu
from jax.experimental.pallas import tpu_sc as plsc

BATCH = 16384
DIM = 32
_INFO = plsc.get_sparse_core_info()
NC, NS, L = _INFO.num_cores, _INFO.num_subcores, _INFO.num_lanes
NW = NC * NS
B_PER_W = BATCH // NW
CHUNK = 128
NCHUNK = B_PER_W // CHUNK
NGROUP = B_PER_W // L


def _sc_body(user_hbm, item_hbm, u_hbm, i_hbm, j_hbm,
             out_i_hbm, out_j_hbm,
             idx_u, idx_i, idx_j, u_rows, i_rows, j_rows,
             out_i_v, out_j_v, sem):
    wid = lax.axis_index("s") * NC + lax.axis_index("c")
    base_row = wid * NCHUNK

    pltpu.sync_copy(u_hbm.at[pl.ds(base_row, NCHUNK)], idx_u)
    pltpu.sync_copy(i_hbm.at[pl.ds(base_row, NCHUNK)], idx_i)
    pltpu.sync_copy(j_hbm.at[pl.ds(base_row, NCHUNK)], idx_j)

    copies = []
    for k in range(NCHUNK):
        sl = pl.ds(k * CHUNK, CHUNK)
        copies.append(pltpu.make_async_copy(
            user_hbm.at[idx_u.at[k]], u_rows.at[sl], sem))
        copies.append(pltpu.make_async_copy(
            item_hbm.at[idx_i.at[k]], i_rows.at[sl], sem))
        copies.append(pltpu.make_async_copy(
            item_hbm.at[idx_j.at[k]], j_rows.at[sl], sem))
    for c in copies:
        c.start()
    for c in copies:
        c.wait()

    lane = lax.iota(jnp.int32, L)
    zeros = jnp.zeros((L,), jnp.float32)

    def group(g, _):
        rows = lane + g * L
        acc_i = zeros
        acc_j = zeros
        for d in range(DIM):
            col = jnp.full((L,), d, jnp.int32)
            uv = plsc.load_gather(u_rows, [rows, col])
            iv = plsc.load_gather(i_rows, [rows, col])
            jv = plsc.load_gather(j_rows, [rows, col])
            acc_i = acc_i + uv * iv
            acc_j = acc_j + uv * jv
        out_i_v[pl.ds(g * L, L)] = acc_i
        out_j_v[pl.ds(g * L, L)] = acc_j
        return _

    lax.fori_loop(0, NGROUP, group, None)

    out_base = wid * B_PER_W
    pltpu.sync_copy(out_i_v, out_i_hbm.at[pl.ds(out_base, B_PER_W)])
    pltpu.sync_copy(out_j_v, out_j_hbm.at[pl.ds(out_base, B_PER_W)])


@jax.jit
def _bpr_sc(user_embd, item_embd, u2d, i2d, j2d):
    mesh = plsc.VectorSubcoreMesh(core_axis_name="c", subcore_axis_name="s")
    out = jax.ShapeDtypeStruct((BATCH,), jnp.float32)
    f = pl.kernel(
        _sc_body,
        out_type=(out, out),
        mesh=mesh,
        scratch_types=[
            pltpu.VMEM((NCHUNK, CHUNK), jnp.int32),
            pltpu.VMEM((NCHUNK, CHUNK), jnp.int32),
            pltpu.VMEM((NCHUNK, CHUNK), jnp.int32),
            pltpu.VMEM((B_PER_W, DIM), jnp.float32),
            pltpu.VMEM((B_PER_W, DIM), jnp.float32),
            pltpu.VMEM((B_PER_W, DIM), jnp.float32),
            pltpu.VMEM((B_PER_W,), jnp.float32),
            pltpu.VMEM((B_PER_W,), jnp.float32),
            pltpu.SemaphoreType.DMA,
        ],
        compiler_params=pltpu.CompilerParams(
            needs_layout_passes=False, use_tc_tiling_on_sc=False),
    )
    return f(user_embd, item_embd, u2d, i2d, j2d)


def kernel(user_embd, item_embd, u, i, j):
    u2d = u.astype(jnp.int32).reshape(NW * NCHUNK, CHUNK)
    i2d = i.astype(jnp.int32).reshape(NW * NCHUNK, CHUNK)
    j2d = j.astype(jnp.int32).reshape(NW * NCHUNK, CHUNK)
    return _bpr_sc(user_embd, item_embd, u2d, i2d, j2d)


# speedup vs baseline: 1.4775x; 1.4775x over previous
import jax
import jax.numpy as jnp
from jax import lax
from jax.experimental import pallas as pl
from jax.experimental.pallas import tpu as pltpu
from jax.experimental.pallas import tpu_sc as plsc

BATCH = 16384
DIM = 32
_INFO = plsc.get_sparse_core_info()
NC, NS, L = _INFO.num_cores, _INFO.num_subcores, _INFO.num_lanes
NW = NC * NS
B_PER_W = BATCH // NW
CHUNK = 128
NCHUNK = B_PER_W // CHUNK
GPC = CHUNK // L


def _sc_body(user_hbm, item_hbm, u_hbm, i_hbm, j_hbm, dummy_hbm,
             out_i_hbm, out_j_hbm,
             idx_u, idx_i, idx_j,
             u_b0, i_b0, j_b0, u_b1, i_b1, j_b1,
             out_i_v, out_j_v, sem0, sem1):
    wid = lax.axis_index("s") * NC + lax.axis_index("c")
    base_row = wid * NCHUNK

    pltpu.sync_copy(u_hbm.at[pl.ds(base_row, NCHUNK)], idx_u)
    pltpu.sync_copy(i_hbm.at[pl.ds(base_row, NCHUNK)], idx_i)
    pltpu.sync_copy(j_hbm.at[pl.ds(base_row, NCHUNK)], idx_j)

    lane = lax.iota(jnp.int32, L)
    bufs = ((u_b0, i_b0, j_b0, sem0), (u_b1, i_b1, j_b1, sem1))

    def issue_chunk(k):
        ub, ib, jb, sem = bufs[k % 2]

        def body(v, _):
            chunk_vec = jnp.full((L,), k, jnp.int32)
            pos = lane + v * L
            iv_u = plsc.load_gather(idx_u, [chunk_vec, pos])
            iv_i = plsc.load_gather(idx_i, [chunk_vec, pos])
            iv_j = plsc.load_gather(idx_j, [chunk_vec, pos])
            base = v * L
            for l in range(L):
                dst = pl.ds(base + l, 1)
                pltpu.make_async_copy(
                    user_hbm.at[pl.ds(iv_u[l], 1)], ub.at[dst], sem).start()
                pltpu.make_async_copy(
                    item_hbm.at[pl.ds(iv_i[l], 1)], ib.at[dst], sem).start()
                pltpu.make_async_copy(
                    item_hbm.at[pl.ds(iv_j[l], 1)], jb.at[dst], sem).start()
            return _

        lax.fori_loop(0, GPC, body, None)

    def drain_chunk(k):
        ub, ib, jb, sem = bufs[k % 2]
        pltpu.make_async_copy(dummy_hbm, ub, sem).wait()
        pltpu.make_async_copy(dummy_hbm, ib, sem).wait()
        pltpu.make_async_copy(dummy_hbm, jb, sem).wait()

    zeros = jnp.zeros((L,), jnp.float32)

    def compute_chunk(k):
        ub, ib, jb, _ = bufs[k % 2]

        def body(g, _):
            rows = g * L + lane
            acc_i = zeros
            acc_j = zeros
            for d in range(DIM):
                col = jnp.full((L,), d, jnp.int32)
                uv = plsc.load_gather(ub, [rows, col])
                iv = plsc.load_gather(ib, [rows, col])
                jv = plsc.load_gather(jb, [rows, col])
                acc_i = acc_i + uv * iv
                acc_j = acc_j + uv * jv
            out_i_v[pl.ds(k * CHUNK + g * L, L)] = acc_i
            out_j_v[pl.ds(k * CHUNK + g * L, L)] = acc_j
            return _

        lax.fori_loop(0, GPC, body, None)

    issue_chunk(0)
    for k in range(NCHUNK):
        if k + 1 < NCHUNK:
            issue_chunk(k + 1)
        drain_chunk(k)
        compute_chunk(k)

    out_base = wid * B_PER_W
    pltpu.sync_copy(out_i_v, out_i_hbm.at[pl.ds(out_base, B_PER_W)])
    pltpu.sync_copy(out_j_v, out_j_hbm.at[pl.ds(out_base, B_PER_W)])


@jax.jit
def _bpr_sc(user_embd, item_embd, u2d, i2d, j2d, dummy):
    mesh = plsc.VectorSubcoreMesh(core_axis_name="c", subcore_axis_name="s")
    out = jax.ShapeDtypeStruct((BATCH,), jnp.float32)
    idx_t = pltpu.VMEM((NCHUNK, CHUNK), jnp.int32)
    buf_t = pltpu.VMEM((CHUNK, DIM), jnp.float32)
    f = pl.kernel(
        _sc_body,
        out_type=(out, out),
        mesh=mesh,
        scratch_types=[
            idx_t, idx_t, idx_t,
            buf_t, buf_t, buf_t, buf_t, buf_t, buf_t,
            pltpu.VMEM((B_PER_W,), jnp.float32),
            pltpu.VMEM((B_PER_W,), jnp.float32),
            pltpu.SemaphoreType.DMA,
            pltpu.SemaphoreType.DMA,
        ],
        compiler_params=pltpu.CompilerParams(needs_layout_passes=False),
    )
    return f(user_embd, item_embd, u2d, i2d, j2d, dummy)


def kernel(user_embd, item_embd, u, i, j):
    u2d = u.astype(jnp.int32).reshape(NW * NCHUNK, CHUNK)
    i2d = i.astype(jnp.int32).reshape(NW * NCHUNK, CHUNK)
    j2d = j.astype(jnp.int32).reshape(NW * NCHUNK, CHUNK)
    dummy = jnp.zeros((CHUNK, DIM), jnp.float32)
    return _bpr_sc(user_embd, item_embd, u2d, i2d, j2d, dummy)
